# Initial kernel scaffold; baseline (speedup 1.0000x reference)
import functools

import jax
import jax.numpy as jnp
from jax import lax
from jax.experimental import pallas as pl
from jax.experimental.pallas import tpu as pltpu
from jax.experimental.pallas import tpu_sc as plsc

VOCAB = 1000000
EMB = 64
TOT = 16384 * 50
NC, NS = 2, 16
NW = NC * NS
PER_W = TOT // NW
CHUNK = 128
NCHUNK = PER_W // CHUNK

_mesh = plsc.VectorSubcoreMesh(core_axis_name="c", subcore_axis_name="s")


@functools.partial(
    pl.kernel,
    mesh=_mesh,
    out_type=jax.ShapeDtypeStruct((TOT, EMB), jnp.float32),
    scratch_types=[
        pltpu.VMEM((NCHUNK, CHUNK), jnp.int32),
        pltpu.VMEM((CHUNK, EMB), jnp.float32),
        pltpu.SemaphoreType.DMA,
    ],
)
def _gather_kernel(idx_hbm, table_hbm, out_hbm, idx_v, rows_v, sem):
    wid = lax.axis_index("s") * NC + lax.axis_index("c")
    base = wid * PER_W
    pltpu.sync_copy(idx_hbm.at[pl.ds(wid * NCHUNK, NCHUNK)], idx_v)

    def body(g, _):
        pltpu.async_copy(table_hbm.at[idx_v.at[g]], rows_v, sem).wait()
        pltpu.sync_copy(rows_v, out_hbm.at[pl.ds(base + g * CHUNK, CHUNK)])
        return ()

    lax.fori_loop(0, NCHUNK, body, ())


def kernel(input, table):
    idx = input.reshape(NW * NCHUNK, CHUNK).astype(jnp.int32)
    out = _gather_kernel(idx, table)
    return out.reshape(input.shape[0], input.shape[1], EMB)


# speedup vs baseline: 1.6840x; 1.6840x over previous
import functools

import jax
import jax.numpy as jnp
from jax import lax
from jax.experimental import pallas as pl
from jax.experimental.pallas import tpu as pltpu
from jax.experimental.pallas import tpu_sc as plsc

VOCAB = 1000000
EMB = 64
TOT = 16384 * 50
NC, NS = 2, 16
NW = NC * NS
PER_W = TOT // NW
CHUNK = 128
NCHUNK = PER_W // CHUNK

_mesh = plsc.VectorSubcoreMesh(core_axis_name="c", subcore_axis_name="s")


@functools.partial(
    pl.kernel,
    mesh=_mesh,
    out_type=jax.ShapeDtypeStruct((TOT, EMB), jnp.float32),
    compiler_params=pltpu.CompilerParams(use_tc_tiling_on_sc=False),
    scratch_types=[
        pltpu.VMEM((NCHUNK, CHUNK), jnp.int32),
        pltpu.VMEM((CHUNK, EMB), jnp.float32),
        pltpu.SemaphoreType.DMA,
    ],
)
def _gather_kernel(idx_hbm, table_hbm, out_hbm, idx_v, rows_v, sem):
    wid = lax.axis_index("s") * NC + lax.axis_index("c")
    base = wid * PER_W
    pltpu.sync_copy(idx_hbm.at[pl.ds(wid * NCHUNK, NCHUNK)], idx_v)

    def body(g, _):
        pltpu.async_copy(table_hbm.at[idx_v.at[g]], rows_v, sem).wait()
        pltpu.sync_copy(rows_v, out_hbm.at[pl.ds(base + g * CHUNK, CHUNK)])
        return ()

    lax.fori_loop(0, NCHUNK, body, ())


def kernel(input, table):
    idx = input.reshape(NW * NCHUNK, CHUNK).astype(jnp.int32)
    out = _gather_kernel(idx, table)
    return out.reshape(input.shape[0], input.shape[1], EMB)


# speedup vs baseline: 1.8726x; 1.1120x over previous
import functools

import jax
import jax.numpy as jnp
from jax import lax
from jax.experimental import pallas as pl
from jax.experimental.pallas import tpu as pltpu
from jax.experimental.pallas import tpu_sc as plsc

VOCAB = 1000000
EMB = 64
TOT = 16384 * 50
NC, NS = 2, 16
NW = NC * NS
PER_W = TOT // NW
CHUNK = 128
NCHUNK = PER_W // CHUNK
K = 4
GROUP = K * CHUNK
NGROUP = NCHUNK // K

_mesh = plsc.VectorSubcoreMesh(core_axis_name="c", subcore_axis_name="s")


@functools.partial(
    pl.kernel,
    mesh=_mesh,
    out_type=jax.ShapeDtypeStruct((TOT, EMB), jnp.float32),
    compiler_params=pltpu.CompilerParams(use_tc_tiling_on_sc=False),
    scratch_types=[
        pltpu.VMEM((NCHUNK, CHUNK), jnp.int32),
        pltpu.VMEM((GROUP, EMB), jnp.float32),
        pltpu.VMEM((GROUP, EMB), jnp.float32),
        pltpu.SemaphoreType.DMA,
        pltpu.SemaphoreType.DMA,
        pltpu.SemaphoreType.DMA,
    ],
)
def _gather_kernel(idx_hbm, table_hbm, out_hbm, idx_v, rows0, rows1,
                   gsem, osem0, osem1):
    wid = lax.axis_index("s") * NC + lax.axis_index("c")
    base = wid * PER_W
    pltpu.sync_copy(idx_hbm.at[pl.ds(wid * NCHUNK, NCHUNK)], idx_v)

    def do_group(g, rows, osem, reuse):
        @pl.when(reuse)
        def _():
            pltpu.make_async_copy(rows, out_hbm.at[pl.ds(base, GROUP)], osem).wait()
        descs = [
            pltpu.make_async_copy(
                table_hbm.at[idx_v.at[g * K + k]],
                rows.at[pl.ds(k * CHUNK, CHUNK)], gsem)
            for k in range(K)
        ]
        for d in descs:
            d.start()
        for d in descs:
            d.wait()
        pltpu.make_async_copy(
            rows, out_hbm.at[pl.ds(base + g * GROUP, GROUP)], osem).start()

    def body(s, _):
        do_group(2 * s, rows0, osem0, s >= 1)
        do_group(2 * s + 1, rows1, osem1, s >= 1)
        return ()

    lax.fori_loop(0, NGROUP // 2, body, ())
    pltpu.make_async_copy(rows0, out_hbm.at[pl.ds(base, GROUP)], osem0).wait()
    pltpu.make_async_copy(rows1, out_hbm.at[pl.ds(base, GROUP)], osem1).wait()


def kernel(input, table):
    idx = input.reshape(NW * NCHUNK, CHUNK).astype(jnp.int32)
    out = _gather_kernel(idx, table)
    return out.reshape(input.shape[0], input.shape[1], EMB)


# speedup vs baseline: 1.8765x; 1.0021x over previous
import functools

import jax
import jax.numpy as jnp
from jax import lax
from jax.experimental import pallas as pl
from jax.experimental.pallas import tpu as pltpu
from jax.experimental.pallas import tpu_sc as plsc

VOCAB = 1000000
EMB = 64
TOT = 16384 * 50
NC, NS = 2, 16
NW = NC * NS
PER_W = TOT // NW
CHUNK = 128
NCHUNK = PER_W // CHUNK
K = 4
GROUP = K * CHUNK
NGROUP = NCHUNK // K

_mesh = plsc.VectorSubcoreMesh(core_axis_name="c", subcore_axis_name="s")


@functools.partial(
    pl.kernel,
    mesh=_mesh,
    out_type=jax.ShapeDtypeStruct((TOT, EMB), jnp.float32),
    compiler_params=pltpu.CompilerParams(use_tc_tiling_on_sc=False),
    scratch_types=[
        pltpu.VMEM((NCHUNK, CHUNK), jnp.int32),
        pltpu.VMEM((GROUP, EMB), jnp.float32),
        pltpu.VMEM((GROUP, EMB), jnp.float32),
        pltpu.SemaphoreType.DMA,
        pltpu.SemaphoreType.DMA,
        pltpu.SemaphoreType.DMA,
        pltpu.SemaphoreType.DMA,
    ],
)
def _gather_kernel(idx_hbm, table_hbm, out_hbm, idx_v, rows0, rows1,
                   gsem0, gsem1, osem0, osem1):
    wid = lax.axis_index("s") * NC + lax.axis_index("c")
    base = wid * PER_W
    pltpu.sync_copy(idx_hbm.at[pl.ds(wid * NCHUNK, NCHUNK)], idx_v)

    def gather_descs(g, rows, gsem):
        return [
            pltpu.make_async_copy(
                table_hbm.at[idx_v.at[g * K + k]],
                rows.at[pl.ds(k * CHUNK, CHUNK)], gsem)
            for k in range(K)
        ]

    def issue(g, rows, gsem):
        for d in gather_descs(g, rows, gsem):
            d.start()

    def finish(g, rows, gsem, osem):
        for d in gather_descs(g, rows, gsem):
            d.wait()
        pltpu.make_async_copy(
            rows, out_hbm.at[pl.ds(base + g * GROUP, GROUP)], osem).start()

    def wait_out(rows, osem):
        pltpu.make_async_copy(rows, out_hbm.at[pl.ds(base, GROUP)], osem).wait()

    NG2 = NGROUP // 2
    issue(0, rows0, gsem0)

    def body(s, _):
        @pl.when(s >= 1)
        def _():
            wait_out(rows1, osem1)
        issue(2 * s + 1, rows1, gsem1)
        finish(2 * s, rows0, gsem0, osem0)
        @pl.when(s < NG2 - 1)
        def _():
            wait_out(rows0, osem0)
            issue(2 * s + 2, rows0, gsem0)
        finish(2 * s + 1, rows1, gsem1, osem1)
        return ()

    lax.fori_loop(0, NG2, body, ())
    wait_out(rows0, osem0)
    wait_out(rows1, osem1)


def kernel(input, table):
    idx = input.reshape(NW * NCHUNK, CHUNK).astype(jnp.int32)
    out = _gather_kernel(idx, table)
    return out.reshape(input.shape[0], input.shape[1], EMB)


# speedup vs baseline: 1.9578x; 1.0433x over previous
import functools

import jax
import jax.numpy as jnp
from jax import lax
from jax.experimental import pallas as pl
from jax.experimental.pallas import tpu as pltpu
from jax.experimental.pallas import tpu_sc as plsc

VOCAB = 1000000
EMB = 64
TOT = 16384 * 50
NC, NS = 2, 16
NW = NC * NS
PER_W = TOT // NW
CHUNK = 128
NCHUNK = PER_W // CHUNK
K = 4
GROUP = K * CHUNK
NGROUP = NCHUNK // K

_mesh = plsc.VectorSubcoreMesh(core_axis_name="c", subcore_axis_name="s")


@functools.partial(
    pl.kernel,
    mesh=_mesh,
    out_type=jax.ShapeDtypeStruct((TOT, EMB), jnp.float32),
    compiler_params=pltpu.CompilerParams(use_tc_tiling_on_sc=False),
    scratch_types=[
        pltpu.VMEM((NCHUNK, CHUNK), jnp.int32),
        pltpu.VMEM((GROUP, EMB), jnp.float32),
        pltpu.VMEM((GROUP, EMB), jnp.float32),
        pltpu.SemaphoreType.DMA,
        pltpu.SemaphoreType.DMA,
        pltpu.SemaphoreType.DMA,
        pltpu.SemaphoreType.DMA,
    ],
)
def _gather_kernel(idx_hbm, table_hbm, out_hbm, idx_v, rows0, rows1,
                   gsem0, gsem1, osem0, osem1):
    wid = lax.axis_index("s") * NC + lax.axis_index("c")
    base = wid * PER_W
    pltpu.sync_copy(idx_hbm.at[pl.ds(wid * NCHUNK, NCHUNK)], idx_v)

    def gather_descs(g, rows, gsem):
        return [
            pltpu.make_async_copy(
                table_hbm.at[idx_v.at[g * K + k]],
                rows.at[pl.ds(k * CHUNK, CHUNK)], gsem)
            for k in range(K)
        ]

    def issue(g, rows, gsem):
        for d in gather_descs(g, rows, gsem):
            d.start()

    def finish(g, rows, gsem, osem):
        for d in gather_descs(g, rows, gsem):
            d.wait()
        pltpu.make_async_copy(
            rows, out_hbm.at[pl.ds(base + g * GROUP, GROUP)], osem).start()

    def wait_out(rows, osem):
        pltpu.make_async_copy(rows, out_hbm.at[pl.ds(base, GROUP)], osem).wait()

    NG2 = NGROUP // 2
    issue(0, rows0, gsem0)

    def body(s, _):
        @pl.when(s >= 1)
        def _():
            wait_out(rows1, osem1)
        issue(2 * s + 1, rows1, gsem1)
        finish(2 * s, rows0, gsem0, osem0)
        @pl.when(s < NG2 - 1)
        def _():
            wait_out(rows0, osem0)
            issue(2 * s + 2, rows0, gsem0)
        finish(2 * s + 1, rows1, gsem1, osem1)
        return ()

    lax.fori_loop(0, NG2, body, ())
    wait_out(rows0, osem0)
    wait_out(rows1, osem1)


def kernel(input, table):
    idx = input.T.reshape(NW * NCHUNK, CHUNK).astype(jnp.int32)
    out = _gather_kernel(idx, table)
    return out.reshape(input.shape[1], input.shape[0], EMB).transpose(1, 0, 2)


# speedup vs baseline: 2.3220x; 1.1861x over previous
import functools

import jax
import jax.numpy as jnp
from jax import lax
from jax.experimental import pallas as pl
from jax.experimental.pallas import tpu as pltpu
from jax.experimental.pallas import tpu_sc as plsc

VOCAB = 1000000
EMB = 64
BATCH = 16384
HIST = 50
TOT = BATCH * HIST
NC, NS = 2, 16
NW = NC * NS
PER_W = TOT // NW
CHUNK = 128
NCHUNK = PER_W // CHUNK
K = 4
GROUP = K * CHUNK
NGROUP = NCHUNK // K
NBT = BATCH // CHUNK
CPAD = 133

_mesh = plsc.VectorSubcoreMesh(core_axis_name="c", subcore_axis_name="s")


@functools.partial(
    pl.kernel,
    mesh=_mesh,
    out_type=jax.ShapeDtypeStruct((HIST, 8, NBT, 8, CHUNK), jnp.float32),
    compiler_params=pltpu.CompilerParams(
        use_tc_tiling_on_sc=False, needs_layout_passes=False),
    scratch_types=[
        pltpu.VMEM((NCHUNK, CHUNK), jnp.int32),
        pltpu.VMEM((GROUP, EMB), jnp.float32),
        pltpu.VMEM((GROUP, EMB), jnp.float32),
        pltpu.VMEM((8, 8, CPAD), jnp.float32),
        pltpu.VMEM((8, 8, CPAD), jnp.float32),
        pltpu.SemaphoreType.DMA,
        pltpu.SemaphoreType.DMA,
        pltpu.SemaphoreType.DMA,
        pltpu.SemaphoreType.DMA,
    ],
)
def _gather_kernel(idx_hbm, table_hbm, out_hbm, idx_v, rows0, rows1,
                   t0, t1, gsem0, gsem1, osem0, osem1):
    wid = lax.axis_index("s") * NC + lax.axis_index("c")
    base_blk = wid * NCHUNK
    pltpu.sync_copy(idx_hbm.at[pl.ds(base_blk, NCHUNK)], idx_v)

    lane = lax.iota(jnp.int32, 16)
    dts = [((d0 + lane) >> 3).astype(jnp.int32) for d0 in range(0, EMB, 16)]
    rs = [((d0 + lane) & 7).astype(jnp.int32) for d0 in range(0, EMB, 16)]
    zeros16 = jnp.zeros((16,), jnp.int32)

    def gather_descs(g, rows, gsem):
        return [
            pltpu.make_async_copy(
                table_hbm.at[idx_v.at[g * K + k]],
                rows.at[pl.ds(k * CHUNK, CHUNK)], gsem)
            for k in range(K)
        ]

    def issue(g, rows, gsem):
        for d in gather_descs(g, rows, gsem):
            d.start()

    def out_desc(blk, t, osem):
        h = blk // NBT
        bt = blk - h * NBT
        return pltpu.make_async_copy(
            t.at[:, :, pl.ds(0, CHUNK)], out_hbm.at[h, :, bt], osem)

    def transpose_block(rows, k, t):
        def body(c, _):
            cv = zeros16 + c
            for j in range(EMB // 16):
                v = rows[k * CHUNK + c, pl.ds(j * 16, 16)]
                plsc.store_scatter(t, [dts[j], rs[j], cv], v)
            return ()

        lax.fori_loop(0, CHUNK, body, ())

    def finish(g, rows, gsem, osem, first):
        for d in gather_descs(g, rows, gsem):
            d.wait()
        for k in range(K):
            t = (t0, t1)[k % 2]
            tsem = (osem0, osem1)[k % 2]
            @pl.when(jnp.logical_not(first) | (k >= 2))
            def _():
                out_desc(0, t, tsem).wait()
            transpose_block(rows, k, t)
            out_desc(base_blk + g * K + k, t, tsem).start()

    NG2 = NGROUP // 2
    issue(0, rows0, gsem0)

    def body(s, _):
        issue(2 * s + 1, rows1, gsem1)
        finish(2 * s, rows0, gsem0, osem0, s == 0)

        @pl.when(s < NG2 - 1)
        def _():
            issue(2 * s + 2, rows0, gsem0)
        finish(2 * s + 1, rows1, gsem1, osem1, jnp.bool_(False))
        return ()

    lax.fori_loop(0, NG2, body, ())
    out_desc(0, t0, osem0).wait()
    out_desc(0, t1, osem1).wait()


def kernel(input, table):
    idx = input.T.reshape(NW * NCHUNK, CHUNK).astype(jnp.int32)
    out5 = _gather_kernel(idx, table)
    return out5.transpose(2, 4, 0, 1, 3).reshape(BATCH, HIST, EMB)


# speedup vs baseline: 3.1119x; 1.3402x over previous
import functools

import jax
import jax.numpy as jnp
from jax import lax
from jax.experimental import pallas as pl
from jax.experimental.pallas import tpu as pltpu
from jax.experimental.pallas import tpu_sc as plsc

VOCAB = 1000000
EMB = 64
BATCH = 16384
HIST = 50
TOT = BATCH * HIST
NC, NS = 2, 16
NW = NC * NS
PER_W = TOT // NW
CHUNK = 128
NCHUNK = PER_W // CHUNK
K = 4
GROUP = K * CHUNK
NGROUP = NCHUNK // K
NBT = BATCH // CHUNK
CPAD = 133

_mesh = plsc.VectorSubcoreMesh(core_axis_name="c", subcore_axis_name="s")


@functools.partial(
    pl.kernel,
    mesh=_mesh,
    out_type=jax.ShapeDtypeStruct((HIST, 8, NBT, 8, CHUNK), jnp.float32),
    compiler_params=pltpu.CompilerParams(
        use_tc_tiling_on_sc=False, needs_layout_passes=False),
    scratch_types=[
        pltpu.VMEM((NCHUNK, CHUNK), jnp.int32),
        pltpu.VMEM((GROUP, EMB), jnp.float32),
        pltpu.VMEM((GROUP, EMB), jnp.float32),
        pltpu.VMEM((8, 8, CPAD), jnp.float32),
        pltpu.VMEM((8, 8, CPAD), jnp.float32),
        pltpu.SemaphoreType.DMA,
        pltpu.SemaphoreType.DMA,
        pltpu.SemaphoreType.DMA,
        pltpu.SemaphoreType.DMA,
    ],
)
def _gather_kernel(idx_hbm, table_hbm, out_hbm, idx_v, rows0, rows1,
                   t0, t1, gsem0, gsem1, osem0, osem1):
    wid = lax.axis_index("s") * NC + lax.axis_index("c")
    base_blk = wid * NCHUNK
    pltpu.sync_copy(idx_hbm.at[pl.ds(base_blk, NCHUNK)], idx_v)

    lane = lax.iota(jnp.int32, 16)
    dts = [((d0 + lane) >> 3).astype(jnp.int32) for d0 in range(0, EMB, 16)]
    rs = [((d0 + lane) & 7).astype(jnp.int32) for d0 in range(0, EMB, 16)]
    zeros16 = jnp.zeros((16,), jnp.int32)

    def gather_descs(g, rows, gsem):
        return [
            pltpu.make_async_copy(
                table_hbm.at[idx_v.at[g * K + k]],
                rows.at[pl.ds(k * CHUNK, CHUNK)], gsem)
            for k in range(K)
        ]

    def issue(g, rows, gsem):
        for d in gather_descs(g, rows, gsem):
            d.start()

    def out_desc(blk, t, osem):
        h = blk // NBT
        bt = blk - h * NBT
        return pltpu.make_async_copy(
            t.at[:, :, pl.ds(0, CHUNK)], out_hbm.at[h, :, bt], osem)

    def transpose_block(rows, k, t):
        @plsc.parallel_loop(0, CHUNK, 1, unroll=4)
        def body(c):
            cv = zeros16 + c
            for j in range(EMB // 16):
                v = rows[k * CHUNK + c, pl.ds(j * 16, 16)]
                plsc.store_scatter(t, [dts[j], rs[j], cv], v)

    def finish(g, rows, gsem, osem, first):
        for d in gather_descs(g, rows, gsem):
            d.wait()
        for k in range(K):
            t = (t0, t1)[k % 2]
            tsem = (osem0, osem1)[k % 2]
            @pl.when(jnp.logical_not(first) | (k >= 2))
            def _():
                out_desc(0, t, tsem).wait()
            transpose_block(rows, k, t)
            out_desc(base_blk + g * K + k, t, tsem).start()

    NG2 = NGROUP // 2
    issue(0, rows0, gsem0)

    def body(s, _):
        issue(2 * s + 1, rows1, gsem1)
        finish(2 * s, rows0, gsem0, osem0, s == 0)

        @pl.when(s < NG2 - 1)
        def _():
            issue(2 * s + 2, rows0, gsem0)
        finish(2 * s + 1, rows1, gsem1, osem1, jnp.bool_(False))
        return ()

    lax.fori_loop(0, NG2, body, ())
    out_desc(0, t0, osem0).wait()
    out_desc(0, t1, osem1).wait()


def kernel(input, table):
    idx = input.T.reshape(NW * NCHUNK, CHUNK).astype(jnp.int32)
    out5 = _gather_kernel(idx, table)
    return out5.transpose(2, 4, 0, 1, 3).reshape(BATCH, HIST, EMB)
